# baseline (speedup 1.0000x reference)
import functools

import jax
import jax.numpy as jnp
from jax import lax
from jax.experimental import pallas as pl
from jax.experimental.pallas import tpu as pltpu
from jax.experimental.pallas import tpu_sc as plsc

N = 10000
E = 320000
D = 128
NLAYERS = 2
NSUB = 16
LANES = 16
CHUNK = 80
NCHUNK = E // (NSUB * CHUNK)
EPT = E // NSUB
ROWS_T = 624
ROWS_LAST = 640
FB = 8
NBUF = 2
PAGE = 10
NPAGE = NCHUNK // PAGE

_MESH = plsc.VectorSubcoreMesh(core_axis_name="c", subcore_axis_name="s")
_SC_PARAMS = pltpu.CompilerParams(use_tc_tiling_on_sc=False,
                                  needs_layout_passes=False)


def _splat(val, dtype=jnp.float32):
    return jnp.zeros((LANES,), dtype) + val


@functools.partial(
    pl.kernel,
    out_type=jax.ShapeDtypeStruct((NLAYERS, N, D), jnp.float32),
    mesh=_MESH,
    compiler_params=_SC_PARAMS,
    scratch_types=[
        pltpu.VMEM((2, PAGE, CHUNK), jnp.int32),
        pltpu.VMEM((2, PAGE, CHUNK), jnp.int32),
        pltpu.VMEM((2, PAGE * CHUNK), jnp.float32),
        pltpu.VMEM((NBUF, CHUNK), jnp.int32),
        pltpu.VMEM((CHUNK,), jnp.float32),
        pltpu.VMEM((N,), jnp.float32),
        pltpu.VMEM((NBUF, CHUNK, D), jnp.bfloat16),
        pltpu.VMEM((NBUF, CHUNK, D), jnp.float32),
        pltpu.VMEM((D,), jnp.float32),
        pltpu.VMEM((FB, D), jnp.float32),
        pltpu.VMEM((ROWS_LAST,), jnp.float32),
        pltpu.VMEM_SHARED((N, D), jnp.float32),
        pltpu.VMEM_SHARED((N,), jnp.float32),
        pltpu.SemaphoreType.DMA((2,)),
        pltpu.SemaphoreType.DMA((NBUF,)),
        pltpu.SemaphoreType.DMA((NBUF,)),
    ],
)
def _sc_cascade(h0_hbm, h1_hbm, rows_hbm, cols_hbm, ew_hbm, b_hbm,
                out_hbm, ridx_v, cidx_v, ew_v, cring, norm_v, dis_v,
                gbuf, sbuf, b_v, fbuf, zero_v, acc_sh, deg_sh,
                sem_p, sem_g, sem_s):
    c = lax.axis_index("c")
    t = lax.axis_index("s")
    r0 = t * ROWS_T
    tile = c * NSUB + t

    def fire_page(p, par):
        pltpu.async_copy(rows_hbm.at[pl.ds(tile * NCHUNK + p * PAGE, PAGE)],
                         ridx_v.at[par], sem_p.at[par])
        pltpu.async_copy(cols_hbm.at[pl.ds(tile * NCHUNK + p * PAGE, PAGE)],
                         cidx_v.at[par], sem_p.at[par])
        pltpu.async_copy(
            ew_hbm.at[pl.ds(tile * EPT + p * PAGE * CHUNK, PAGE * CHUNK)],
            ew_v.at[par], sem_p.at[par])

    def wait_page(p, par):
        pltpu.make_async_copy(
            rows_hbm.at[pl.ds(tile * NCHUNK + p * PAGE, PAGE)],
            ridx_v.at[par], sem_p.at[par]).wait()
        pltpu.make_async_copy(
            cols_hbm.at[pl.ds(tile * NCHUNK + p * PAGE, PAGE)],
            cidx_v.at[par], sem_p.at[par]).wait()
        pltpu.make_async_copy(
            ew_hbm.at[pl.ds(tile * EPT + p * PAGE * CHUNK, PAGE * CHUNK)],
            ew_v.at[par], sem_p.at[par]).wait()

    def wait_deg_scatter(b):
        pltpu.make_async_copy(ew_v.at[0, pl.ds(0, CHUNK)],
                              deg_sh.at[cring.at[b]], sem_s.at[b]).wait()

    fire_page(0, 0)

    @pl.loop(0, FB)
    def _(i):
        for j in range(D // LANES):
            fbuf[i, pl.ds(j * LANES, LANES)] = jnp.zeros((LANES,), jnp.float32)

    @pl.loop(0, ROWS_LAST, step=LANES)
    def _(i):
        zero_v[pl.ds(i, LANES)] = jnp.zeros((LANES,), jnp.float32)

    @pl.loop(0, ROWS_T, step=FB)
    def _(i):
        pltpu.sync_copy(fbuf, acc_sh.at[pl.ds(r0 + i, FB)])

    @pl.when(t < NSUB - 1)
    def _():
        pltpu.sync_copy(zero_v.at[pl.ds(0, ROWS_T)], deg_sh.at[pl.ds(r0, ROWS_T)])

    @pl.when(t == NSUB - 1)
    def _():
        pltpu.sync_copy(zero_v, deg_sh.at[pl.ds(r0, ROWS_LAST)])

        @pl.loop(ROWS_T, ROWS_LAST, step=FB)
        def _(i):
            pltpu.sync_copy(fbuf, acc_sh.at[pl.ds(r0 + i, FB)])

    wait_page(0, 0)
    fire_page(1, 1)
    plsc.subcore_barrier()

    @pl.loop(0, NPAGE)
    def _(p):
        par = p % 2
        nxt = 1 - par

        @pl.when(p > 0)
        def _():
            wait_page(p, par)
            for b in range(NBUF):
                wait_deg_scatter(b)

        @pl.when(jnp.logical_and(p >= 1, p + 1 < NPAGE))
        def _():
            fire_page(p + 1, nxt)

        for kk in range(PAGE):
            b = kk % NBUF
            if kk >= NBUF:
                wait_deg_scatter(b)
            for j in range(CHUNK // LANES):
                sl = pl.ds(j * LANES, LANES)
                cring[b, sl] = cidx_v[par, kk, sl]
            pltpu.async_copy(ew_v.at[par, pl.ds(kk * CHUNK, CHUNK)],
                             deg_sh.at[cring.at[b]], sem_s.at[b], add=True)

    for b in range(NBUF):
        wait_deg_scatter(b)

    plsc.subcore_barrier()

    pltpu.sync_copy(deg_sh, dis_v)

    @plsc.parallel_loop(0, N, step=LANES, unroll=4)
    def _(i):
        sl = pl.ds(i, LANES)
        d = dis_v[sl]
        yi = jnp.int32(0x5F3759DF) - (plsc.bitcast(d, jnp.int32) >> 1)
        y = plsc.bitcast(yi, jnp.float32)
        half = d * 0.5
        y = y * (1.5 - half * y * y)
        y = y * (1.5 - half * y * y)
        y = y * (1.5 - half * y * y)
        dis_v[sl] = jnp.where(d > 0, y, 0.0)

    fire_page(0, 0)
    wait_page(0, 0)
    fire_page(1, 1)

    def edge_phase(h_hbm):
        for b in range(NBUF):
            pltpu.async_copy(h_hbm.at[ridx_v.at[0, b]], gbuf.at[b],
                             sem_g.at[b])

        @pl.loop(0, NPAGE)
        def _(p):
            par = p % 2
            nxt = 1 - par

            @pl.when(jnp.logical_and(p >= 1, p + 1 < NPAGE))
            def _():
                fire_page(p + 1, nxt)

            for kk in range(PAGE):
                b = kk % NBUF
                pltpu.make_async_copy(h_hbm.at[ridx_v.at[par, kk]],
                                      gbuf.at[b], sem_g.at[b]).wait()

                def wait_scatter():
                    pltpu.make_async_copy(sbuf.at[b],
                                          acc_sh.at[cring.at[b]],
                                          sem_s.at[b]).wait()
                if kk < NBUF:
                    @pl.when(p > 0)
                    def _():
                        wait_scatter()
                else:
                    wait_scatter()

                for j in range(CHUNK // LANES):
                    sl = pl.ds(j * LANES, LANES)
                    ri = ridx_v[par, kk, sl]
                    cring[b, sl] = cidx_v[par, kk, sl]
                    disr = plsc.load_gather(dis_v, [ri])
                    ewv = ew_v[par, pl.ds(kk * CHUNK + j * LANES, LANES)]
                    norm_v[sl] = disr * ewv

                @plsc.parallel_loop(0, CHUNK, unroll=8)
                def _(e):
                    w = plsc.load_gather(norm_v, [_splat(e, jnp.int32)])
                    for q in range(D // (2 * LANES)):
                        v = gbuf[b, e, pl.ds(q * 2 * LANES, 2 * LANES)]
                        lo, hi = plsc.unpack(v, format=plsc.PackFormat.INTERLEAVED,
                                             preferred_element_type=jnp.float32)
                        sbuf[b, e, pl.ds(q * 2 * LANES, LANES)] = lo * w
                        sbuf[b, e, pl.ds(q * 2 * LANES + LANES, LANES)] = hi * w

                pltpu.async_copy(sbuf.at[b], acc_sh.at[cring.at[b]],
                                 sem_s.at[b], add=True)

                if kk == PAGE - NBUF:
                    @pl.when(p + 1 < NPAGE)
                    def _():
                        wait_page(p + 1, nxt)

                if kk < PAGE - NBUF:
                    pltpu.async_copy(h_hbm.at[ridx_v.at[par, kk + NBUF]],
                                     gbuf.at[b], sem_g.at[b])
                else:
                    @pl.when(p + 1 < NPAGE)
                    def _():
                        pltpu.async_copy(
                            h_hbm.at[ridx_v.at[nxt, kk - (PAGE - NBUF)]],
                            gbuf.at[b], sem_g.at[b])

        for b in range(NBUF):
            pltpu.make_async_copy(sbuf.at[b], acc_sh.at[cring.at[b]],
                                  sem_s.at[b]).wait()

    @pl.when(c == 0)
    def _():
        edge_phase(h0_hbm)

    @pl.when(c == 1)
    def _():
        edge_phase(h1_hbm)

    plsc.subcore_barrier()

    pltpu.sync_copy(b_hbm.at[c], b_v)

    def flush(nrows):
        @pl.loop(0, nrows, step=FB)
        def _(i):
            pltpu.sync_copy(acc_sh.at[pl.ds(r0 + i, FB)], fbuf)
            for r in range(FB):
                d = plsc.load_gather(dis_v, [_splat(r0 + i + r, jnp.int32)])
                for j in range(D // LANES):
                    sl = pl.ds(j * LANES, LANES)
                    fbuf[r, sl] = jnp.maximum(fbuf[r, sl] * d + b_v[sl], 0.0)
            pltpu.sync_copy(fbuf, out_hbm.at[c, pl.ds(r0 + i, FB)])

    @pl.when(t < NSUB - 1)
    def _():
        flush(ROWS_T)

    @pl.when(t == NSUB - 1)
    def _():
        flush(ROWS_LAST)


_ROWBLK = 1000


def _tc_dense(x, WlT, bl, W01T):
    def body(x_ref, wl_ref, b_ref, w01_ref, hlin_ref, h01_ref):
        xx = x_ref[...]
        hl = jnp.dot(xx, wl_ref[...], preferred_element_type=jnp.float32)
        hlin_ref[...] = jnp.maximum(hl + b_ref[...], 0.0)
        h01_ref[0] = jnp.dot(
            xx, w01_ref[0], preferred_element_type=jnp.float32
        ).astype(jnp.bfloat16)
        h01_ref[1] = jnp.dot(
            xx, w01_ref[1], preferred_element_type=jnp.float32
        ).astype(jnp.bfloat16)

    return pl.pallas_call(
        body,
        grid=(N // _ROWBLK,),
        in_specs=[
            pl.BlockSpec((_ROWBLK, D), lambda i: (i, 0)),
            pl.BlockSpec((D, D), lambda i: (0, 0)),
            pl.BlockSpec((1, D), lambda i: (0, 0)),
            pl.BlockSpec((NLAYERS, D, D), lambda i: (0, 0, 0)),
        ],
        out_specs=[
            pl.BlockSpec((_ROWBLK, D), lambda i: (i, 0)),
            pl.BlockSpec((NLAYERS, _ROWBLK, D), lambda i: (0, i, 0)),
        ],
        out_shape=[
            jax.ShapeDtypeStruct((N, D), jnp.float32),
            jax.ShapeDtypeStruct((NLAYERS, N, D), jnp.bfloat16),
        ],
    )(x, WlT, bl, W01T)


def kernel(x, edge_index, edge_attr, W_lin, b_lin, W0, b0, W1, b1):
    rows = edge_index[:, 0, :].reshape(NLAYERS * NSUB * NCHUNK, CHUNK)
    cols = edge_index[:, 1, :].reshape(NLAYERS * NSUB * NCHUNK, CHUNK)
    ew = edge_attr.reshape(NLAYERS * NSUB * EPT)

    q = jnp.arange(D // 32)[:, None, None] * 32
    i = jnp.arange(LANES)[None, :, None]
    perm = (q + i + jnp.array([0, LANES])[None, None, :]).reshape(D)
    W01T = jnp.stack([W0.T, W1.T])[:, :, perm]

    hlin, h01 = _tc_dense(x, W_lin.T, b_lin.reshape(1, D), W01T)
    out = _sc_cascade(h01[0], h01[1], rows, cols, ew, jnp.stack([b0, b1]))
    return (hlin, out[0], out[1])


# speedup vs baseline: 1.0848x; 1.0848x over previous
import functools

import jax
import jax.numpy as jnp
from jax import lax
from jax.experimental import pallas as pl
from jax.experimental.pallas import tpu as pltpu
from jax.experimental.pallas import tpu_sc as plsc

N = 10000
E = 320000
D = 128
NLAYERS = 2
NSUB = 16
LANES = 16
CHUNK = 80
NCHUNK = E // (NSUB * CHUNK)
EPT = E // NSUB
ROWS_T = 624
ROWS_LAST = 640
FB = 16
NBUF = 2
PAGE = 10
NPAGE = NCHUNK // PAGE

_MESH = plsc.VectorSubcoreMesh(core_axis_name="c", subcore_axis_name="s")
_SC_PARAMS = pltpu.CompilerParams(use_tc_tiling_on_sc=False,
                                  needs_layout_passes=False)


def _splat(val, dtype=jnp.float32):
    return jnp.zeros((LANES,), dtype) + val


@functools.partial(
    pl.kernel,
    out_type=(jax.ShapeDtypeStruct((N, D), jnp.float32),
              jax.ShapeDtypeStruct((N, D), jnp.float32)),
    mesh=_MESH,
    compiler_params=_SC_PARAMS,
    scratch_types=[
        pltpu.VMEM((2, PAGE, CHUNK), jnp.int32),
        pltpu.VMEM((2, PAGE, CHUNK), jnp.int32),
        pltpu.VMEM((2, PAGE * CHUNK), jnp.float32),
        pltpu.VMEM((NBUF, CHUNK), jnp.int32),
        pltpu.VMEM((CHUNK,), jnp.float32),
        pltpu.VMEM((N,), jnp.float32),
        pltpu.VMEM((NBUF, CHUNK, D), jnp.bfloat16),
        pltpu.VMEM((NBUF, CHUNK, D), jnp.float32),
        pltpu.VMEM((D,), jnp.float32),
        pltpu.VMEM((FB, D), jnp.float32),
        pltpu.VMEM((ROWS_LAST,), jnp.float32),
        pltpu.VMEM_SHARED((N, D), jnp.float32),
        pltpu.VMEM_SHARED((N,), jnp.float32),
        pltpu.SemaphoreType.DMA((2,)),
        pltpu.SemaphoreType.DMA((NBUF,)),
        pltpu.SemaphoreType.DMA((NBUF,)),
    ],
)
def _sc_cascade(h0_hbm, h1_hbm, rows_hbm, cols_hbm, ew_hbm, b_hbm,
                out0_hbm, out1_hbm, ridx_v, cidx_v, ew_v, cring, norm_v, dis_v,
                gbuf, sbuf, b_v, fbuf, zero_v, acc_sh, deg_sh,
                sem_p, sem_g, sem_s):
    c = lax.axis_index("c")
    t = lax.axis_index("s")
    r0 = t * ROWS_T
    tile = c * NSUB + t

    def fire_page(p, par):
        pltpu.async_copy(rows_hbm.at[pl.ds(tile * NCHUNK + p * PAGE, PAGE)],
                         ridx_v.at[par], sem_p.at[par])
        pltpu.async_copy(cols_hbm.at[pl.ds(tile * NCHUNK + p * PAGE, PAGE)],
                         cidx_v.at[par], sem_p.at[par])
        pltpu.async_copy(
            ew_hbm.at[pl.ds(tile * EPT + p * PAGE * CHUNK, PAGE * CHUNK)],
            ew_v.at[par], sem_p.at[par])

    def wait_page(p, par):
        pltpu.make_async_copy(
            rows_hbm.at[pl.ds(tile * NCHUNK + p * PAGE, PAGE)],
            ridx_v.at[par], sem_p.at[par]).wait()
        pltpu.make_async_copy(
            cols_hbm.at[pl.ds(tile * NCHUNK + p * PAGE, PAGE)],
            cidx_v.at[par], sem_p.at[par]).wait()
        pltpu.make_async_copy(
            ew_hbm.at[pl.ds(tile * EPT + p * PAGE * CHUNK, PAGE * CHUNK)],
            ew_v.at[par], sem_p.at[par]).wait()

    def wait_deg_scatter(b):
        pltpu.make_async_copy(ew_v.at[0, pl.ds(0, CHUNK)],
                              deg_sh.at[cring.at[b]], sem_s.at[b]).wait()

    fire_page(0, 0)

    @pl.loop(0, FB)
    def _(i):
        for j in range(D // LANES):
            fbuf[i, pl.ds(j * LANES, LANES)] = jnp.zeros((LANES,), jnp.float32)

    @pl.loop(0, ROWS_LAST, step=LANES)
    def _(i):
        zero_v[pl.ds(i, LANES)] = jnp.zeros((LANES,), jnp.float32)

    @pl.loop(0, ROWS_T, step=FB)
    def _(i):
        pltpu.sync_copy(fbuf, acc_sh.at[pl.ds(r0 + i, FB)])

    @pl.when(t < NSUB - 1)
    def _():
        pltpu.sync_copy(zero_v.at[pl.ds(0, ROWS_T)], deg_sh.at[pl.ds(r0, ROWS_T)])

    @pl.when(t == NSUB - 1)
    def _():
        pltpu.sync_copy(zero_v, deg_sh.at[pl.ds(r0, ROWS_LAST)])

        @pl.loop(ROWS_T, ROWS_LAST, step=FB)
        def _(i):
            pltpu.sync_copy(fbuf, acc_sh.at[pl.ds(r0 + i, FB)])

    wait_page(0, 0)
    fire_page(1, 1)
    plsc.subcore_barrier()

    @pl.loop(0, NPAGE)
    def _(p):
        par = p % 2
        nxt = 1 - par

        @pl.when(p > 0)
        def _():
            wait_page(p, par)
            for b in range(NBUF):
                wait_deg_scatter(b)

        @pl.when(jnp.logical_and(p >= 1, p + 1 < NPAGE))
        def _():
            fire_page(p + 1, nxt)

        for kk in range(PAGE):
            b = kk % NBUF
            if kk >= NBUF:
                wait_deg_scatter(b)
            for j in range(CHUNK // LANES):
                sl = pl.ds(j * LANES, LANES)
                cring[b, sl] = cidx_v[par, kk, sl]
            pltpu.async_copy(ew_v.at[par, pl.ds(kk * CHUNK, CHUNK)],
                             deg_sh.at[cring.at[b]], sem_s.at[b], add=True)

    for b in range(NBUF):
        wait_deg_scatter(b)

    plsc.subcore_barrier()

    pltpu.sync_copy(deg_sh, dis_v)

    @plsc.parallel_loop(0, N, step=LANES, unroll=4)
    def _(i):
        sl = pl.ds(i, LANES)
        d = dis_v[sl]
        yi = jnp.int32(0x5F3759DF) - (plsc.bitcast(d, jnp.int32) >> 1)
        y = plsc.bitcast(yi, jnp.float32)
        half = d * 0.5
        y = y * (1.5 - half * y * y)
        y = y * (1.5 - half * y * y)
        y = y * (1.5 - half * y * y)
        dis_v[sl] = jnp.where(d > 0, y, 0.0)

    fire_page(0, 0)
    wait_page(0, 0)
    fire_page(1, 1)

    def edge_phase(h_hbm):
        for b in range(NBUF):
            pltpu.async_copy(h_hbm.at[ridx_v.at[0, b]], gbuf.at[b],
                             sem_g.at[b])

        @pl.loop(0, NPAGE)
        def _(p):
            par = p % 2
            nxt = 1 - par

            @pl.when(jnp.logical_and(p >= 1, p + 1 < NPAGE))
            def _():
                fire_page(p + 1, nxt)

            for kk in range(PAGE):
                b = kk % NBUF
                pltpu.make_async_copy(h_hbm.at[ridx_v.at[par, kk]],
                                      gbuf.at[b], sem_g.at[b]).wait()

                def wait_scatter():
                    pltpu.make_async_copy(sbuf.at[b],
                                          acc_sh.at[cring.at[b]],
                                          sem_s.at[b]).wait()
                if kk < NBUF:
                    @pl.when(p > 0)
                    def _():
                        wait_scatter()
                else:
                    wait_scatter()

                for j in range(CHUNK // LANES):
                    sl = pl.ds(j * LANES, LANES)
                    ri = ridx_v[par, kk, sl]
                    cring[b, sl] = cidx_v[par, kk, sl]
                    disr = plsc.load_gather(dis_v, [ri])
                    ewv = ew_v[par, pl.ds(kk * CHUNK + j * LANES, LANES)]
                    norm_v[sl] = disr * ewv

                @plsc.parallel_loop(0, CHUNK, unroll=8)
                def _(e):
                    w = plsc.load_gather(norm_v, [_splat(e, jnp.int32)])
                    for q in range(D // (2 * LANES)):
                        v = gbuf[b, e, pl.ds(q * 2 * LANES, 2 * LANES)]
                        lo, hi = plsc.unpack(v, format=plsc.PackFormat.INTERLEAVED,
                                             preferred_element_type=jnp.float32)
                        sbuf[b, e, pl.ds(q * 2 * LANES, LANES)] = lo * w
                        sbuf[b, e, pl.ds(q * 2 * LANES + LANES, LANES)] = hi * w

                pltpu.async_copy(sbuf.at[b], acc_sh.at[cring.at[b]],
                                 sem_s.at[b], add=True)

                if kk == PAGE - NBUF:
                    @pl.when(p + 1 < NPAGE)
                    def _():
                        wait_page(p + 1, nxt)

                if kk < PAGE - NBUF:
                    pltpu.async_copy(h_hbm.at[ridx_v.at[par, kk + NBUF]],
                                     gbuf.at[b], sem_g.at[b])
                else:
                    @pl.when(p + 1 < NPAGE)
                    def _():
                        pltpu.async_copy(
                            h_hbm.at[ridx_v.at[nxt, kk - (PAGE - NBUF)]],
                            gbuf.at[b], sem_g.at[b])

        for b in range(NBUF):
            pltpu.make_async_copy(sbuf.at[b], acc_sh.at[cring.at[b]],
                                  sem_s.at[b]).wait()

    @pl.when(c == 0)
    def _():
        edge_phase(h0_hbm)

    @pl.when(c == 1)
    def _():
        edge_phase(h1_hbm)

    plsc.subcore_barrier()

    pltpu.sync_copy(b_hbm.at[c], b_v)

    def flush(nrows, out_ref):
        @pl.loop(0, nrows, step=FB)
        def _(i):
            pltpu.sync_copy(acc_sh.at[pl.ds(r0 + i, FB)], fbuf)
            for r in range(FB):
                d = plsc.load_gather(dis_v, [_splat(r0 + i + r, jnp.int32)])
                for j in range(D // LANES):
                    sl = pl.ds(j * LANES, LANES)
                    fbuf[r, sl] = jnp.maximum(fbuf[r, sl] * d + b_v[sl], 0.0)
            pltpu.sync_copy(fbuf, out_ref.at[pl.ds(r0 + i, FB)])

    def flush_core(out_ref):
        @pl.when(t < NSUB - 1)
        def _():
            flush(ROWS_T, out_ref)

        @pl.when(t == NSUB - 1)
        def _():
            flush(ROWS_LAST, out_ref)

    @pl.when(c == 0)
    def _():
        flush_core(out0_hbm)

    @pl.when(c == 1)
    def _():
        flush_core(out1_hbm)


_ROWBLK = 1000


def _tc_dense(x, WlT, bl, W01T):
    def body(x_ref, wl_ref, b_ref, w01_ref, hlin_ref, h01_ref):
        xx = x_ref[...]
        hl = jnp.dot(xx, wl_ref[...], preferred_element_type=jnp.float32)
        hlin_ref[...] = jnp.maximum(hl + b_ref[...], 0.0)
        xb = xx.astype(jnp.bfloat16)
        h01_ref[0] = jnp.dot(
            xb, w01_ref[0], preferred_element_type=jnp.float32
        ).astype(jnp.bfloat16)
        h01_ref[1] = jnp.dot(
            xb, w01_ref[1], preferred_element_type=jnp.float32
        ).astype(jnp.bfloat16)

    return pl.pallas_call(
        body,
        grid=(N // _ROWBLK,),
        in_specs=[
            pl.BlockSpec((_ROWBLK, D), lambda i: (i, 0)),
            pl.BlockSpec((D, D), lambda i: (0, 0)),
            pl.BlockSpec((1, D), lambda i: (0, 0)),
            pl.BlockSpec((NLAYERS, D, D), lambda i: (0, 0, 0)),
        ],
        out_specs=[
            pl.BlockSpec((_ROWBLK, D), lambda i: (i, 0)),
            pl.BlockSpec((NLAYERS, _ROWBLK, D), lambda i: (0, i, 0)),
        ],
        out_shape=[
            jax.ShapeDtypeStruct((N, D), jnp.float32),
            jax.ShapeDtypeStruct((NLAYERS, N, D), jnp.bfloat16),
        ],
    )(x, WlT, bl, W01T)


def kernel(x, edge_index, edge_attr, W_lin, b_lin, W0, b0, W1, b1):
    rows = edge_index[:, 0, :].reshape(NLAYERS * NSUB * NCHUNK, CHUNK)
    cols = edge_index[:, 1, :].reshape(NLAYERS * NSUB * NCHUNK, CHUNK)
    ew = edge_attr.reshape(NLAYERS * NSUB * EPT)

    q = jnp.arange(D // 32)[:, None, None] * 32
    i = jnp.arange(LANES)[None, :, None]
    perm = (q + i + jnp.array([0, LANES])[None, None, :]).reshape(D)
    W01T = jnp.stack([W0.T, W1.T])[:, :, perm].astype(jnp.bfloat16)

    hlin, h01 = _tc_dense(x, W_lin.T, b_lin.reshape(1, D), W01T)
    out0, out1 = _sc_cascade(h01[0], h01[1], rows, cols, ew,
                             jnp.stack([b0, b1]))
    return (hlin, out0, out1)
